# baseline (speedup 1.0000x reference)
import functools

import jax
import jax.numpy as jnp
from jax import lax
from jax.experimental import pallas as pl
from jax.experimental.pallas import tpu as pltpu
from jax.experimental.pallas import tpu_sc as plsc

N_NODES = 50000
N_EDGES = 800000
P = 8
D = 4 * P
NC = 2
NS = 16
NW = NC * NS
L = 16

EPW = N_EDGES // NW
C = 500
CPAD = 512
NPAIR = EPW // (2 * C)
G = CPAD // L
GB = 128
NGATH = CPAD // GB


def _splat(v):
    return jnp.full((L,), v, dtype=jnp.int32)


def _i32(v):
    return jnp.int32(v)


def _fori(n, body):
    lax.fori_loop(_i32(0), _i32(n), body, _i32(0))


def _sc_body(ptab, ec, wts, ratios, rmw,
             ecva, ecvb, eiva, eivb, ejva, ejvb,
             xiva, xivb, xjva, xjvb, outva, outvb, wv, w8v,
             sem_ga, sem_gb, sem_wb):
    wid = lax.axis_index("s") * NC + lax.axis_index("c")
    iota16 = lax.iota(jnp.int32, L)
    zero16 = jnp.zeros((L,), dtype=jnp.int32)
    c0s, c1s = _splat(0), _splat(1)

    def dec(ecv, eiv, ejv):
        @plsc.parallel_loop(_i32(0), _i32(G), step=_i32(1))
        def _(g):
            o = g * _i32(L)
            e16 = o + iota16
            eiv[pl.ds(o, L)] = plsc.load_gather(ecv, [e16, c0s])
            ejv[pl.ds(o, L)] = plsc.load_gather(ecv, [e16, c1s])

    def gath_issue(eiv, ejv, xiv, xjv, sem):
        cps = []
        for s in range(0, CPAD, GB):
            cps.append(pltpu.async_copy(ptab.at[eiv.at[pl.ds(s, GB)]],
                                        xiv.at[pl.ds(s, GB)], sem))
            cps.append(pltpu.async_copy(ptab.at[ejv.at[pl.ds(s, GB)]],
                                        xjv.at[pl.ds(s, GB)], sem))
        return cps

    def comp(xiv, xjv, outv):
        @plsc.parallel_loop(_i32(0), _i32(G), step=_i32(1))
        def _(g):
            e16 = g * _i32(L) + iota16
            for p in range(P):
                q = 4 * p
                w1 = plsc.load_gather(xiv, [e16, _splat(q)])
                x1 = plsc.load_gather(xiv, [e16, _splat(q + 1)])
                y1 = plsc.load_gather(xiv, [e16, _splat(q + 2)])
                z1 = plsc.load_gather(xiv, [e16, _splat(q + 3)])
                w2 = plsc.load_gather(xjv, [e16, _splat(q)])
                x2 = plsc.load_gather(xjv, [e16, _splat(q + 1)])
                y2 = plsc.load_gather(xjv, [e16, _splat(q + 2)])
                z2 = plsc.load_gather(xjv, [e16, _splat(q + 3)])
                rw = (w1 * w2 + x1 * x2) + (y1 * y2 + z1 * z2)
                rx = (x1 * w2 - w1 * x2) + (z1 * y2 - y1 * z2)
                ry = (y1 * w2 - w1 * y2) + (x1 * z2 - z1 * x2)
                rz = (z1 * w2 - w1 * z2) + (y1 * x2 - x1 * y2)
                plsc.store_scatter(outv, [e16, _splat(q)], rw)
                plsc.store_scatter(outv, [e16, _splat(q + 1)], rx)
                plsc.store_scatter(outv, [e16, _splat(q + 2)], ry)
                plsc.store_scatter(outv, [e16, _splat(q + 3)], rz)

    def wb_issue(outv, base):
        a = pltpu.async_copy(outv.at[pl.ds(0, C)], ratios.at[pl.ds(base, C)],
                             sem_wb)
        b = pltpu.async_copy(wv.at[pl.ds(0, C * P)],
                             rmw.at[pl.ds(base * _i32(P), C * P)], sem_wb)
        return [a, b]

    for r0 in (C, CPAD - 8):
        rr = _i32(r0) + (iota16 >> _i32(1))
        cc = iota16 & _i32(1)
        plsc.store_scatter(ecva, [rr, cc], zero16)
        plsc.store_scatter(ecvb, [rr, cc], zero16)

    pltpu.sync_copy(wts, w8v)
    wvals = plsc.load_gather(w8v, [zero16, iota16 & _i32(7)])

    def wfill(s, carry):
        wv[pl.ds(s * _i32(L), L)] = wvals
        return carry
    _fori(C * P // L, wfill)

    def pair(d, carry):
        base_a = wid * _i32(EPW) + d * _i32(2 * C)
        base_b = base_a + _i32(C)
        pltpu.sync_copy(ec.at[pl.ds(base_a, C)], ecva.at[pl.ds(0, C)])
        pltpu.sync_copy(ec.at[pl.ds(base_b, C)], ecvb.at[pl.ds(0, C)])
        dec(ecva, eiva, ejva)
        dec(ecvb, eivb, ejvb)
        ga = gath_issue(eiva, ejva, xiva, xjva, sem_ga)
        gb = gath_issue(eivb, ejvb, xivb, xjvb, sem_gb)
        for cp in ga:
            cp.wait()
        comp(xiva, xjva, outva)
        wba = wb_issue(outva, base_a)
        for cp in gb:
            cp.wait()
        comp(xivb, xjvb, outvb)
        wbb = wb_issue(outvb, base_b)
        for cp in wba + wbb:
            cp.wait()
        return carry
    _fori(NPAIR, pair)


@functools.partial(
    pl.kernel,
    out_type=(jax.ShapeDtypeStruct((N_EDGES, D), jnp.float32),
              jax.ShapeDtypeStruct((N_EDGES * P,), jnp.float32)),
    mesh=plsc.VectorSubcoreMesh(core_axis_name="c", subcore_axis_name="s",
                                num_cores=NC, num_subcores=NS),
    compiler_params=pltpu.CompilerParams(needs_layout_passes=False,
                                         use_tc_tiling_on_sc=False),
    scratch_types=[
        pltpu.VMEM((CPAD, 2), jnp.int32),
        pltpu.VMEM((CPAD, 2), jnp.int32),
        pltpu.VMEM((CPAD,), jnp.int32),
        pltpu.VMEM((CPAD,), jnp.int32),
        pltpu.VMEM((CPAD,), jnp.int32),
        pltpu.VMEM((CPAD,), jnp.int32),
        pltpu.VMEM((CPAD, D), jnp.float32),
        pltpu.VMEM((CPAD, D), jnp.float32),
        pltpu.VMEM((CPAD, D), jnp.float32),
        pltpu.VMEM((CPAD, D), jnp.float32),
        pltpu.VMEM((CPAD, D), jnp.float32),
        pltpu.VMEM((CPAD, D), jnp.float32),
        pltpu.VMEM((C * P,), jnp.float32),
        pltpu.VMEM((1, P), jnp.float32),
        pltpu.SemaphoreType.DMA,
        pltpu.SemaphoreType.DMA,
        pltpu.SemaphoreType.DMA,
    ],
)
def _quat_edges_sc(ptab, ec, wts, ratios, rmw, *scratch):
    _sc_body(ptab, ec, wts, ratios, rmw, *scratch)


def kernel(particles, weights, edges):
    ec = edges.astype(jnp.int32)
    ptab = particles.astype(jnp.float32).reshape(N_NODES, D)
    ratios, rmw = _quat_edges_sc(ptab, ec, weights.astype(jnp.float32))
    return ratios.reshape(N_EDGES, P, 4), rmw.reshape(N_EDGES, P)
